# baseline (speedup 1.0000x reference)
import functools

import jax
import jax.numpy as jnp
from jax import lax
from jax.experimental import pallas as pl
from jax.experimental.pallas import tpu as pltpu
from jax.experimental.pallas import tpu_sc as plsc

NC = 2
NS = 16
L = 16
NW = NC * NS

B = 128
DC = 64
W2C = 2 * DC
NP = 10240
RPT = NP // NS
ZR = 40


def _make_sc_pass(nbk, nch, with_cnt):
  mesh = plsc.VectorSubcoreMesh(core_axis_name="c", subcore_axis_name="s",
                                num_cores=NC, num_subcores=NS)
  out_type = [jax.ShapeDtypeStruct((NC, nch, NP, W2C), jnp.float32)]
  scratch = [
      pltpu.VMEM((nbk, B), jnp.int32),
      pltpu.VMEM((nbk, B), jnp.int32),
      pltpu.VMEM((B,), jnp.int32),
      pltpu.VMEM((B, W2C), jnp.float32),
      pltpu.VMEM((B, L), jnp.float32),
      pltpu.VMEM((ZR, W2C), jnp.float32),
      pltpu.VMEM_SHARED((NP, W2C), jnp.float32),
      pltpu.SemaphoreType.DMA,
  ]
  if with_cnt:
    out_type.append(jax.ShapeDtypeStruct((NC, NP, W2C), jnp.float32))

  @functools.partial(pl.kernel, out_type=tuple(out_type), mesh=mesh,
                     scratch_types=scratch)
  def body(table_h, srcv_h, dstv_h, ubv_h, *rest):
    if with_cnt:
      sa_out, cnt_out, src_v, dst_v, idxb, g, ubb, zbuf, acc, sem = rest
    else:
      sa_out, src_v, dst_v, idxb, g, ubb, zbuf, acc, sem = rest
    cid = lax.axis_index("c")
    sid = lax.axis_index("s")
    wid = sid * NC + cid
    r0 = sid * RPT

    zv = jnp.zeros((L,), jnp.float32)

    def zrow(i, _):
      for k in range(W2C // L):
        zbuf[i, pl.ds(k * L, L)] = zv
      return 0
    lax.fori_loop(0, ZR, zrow, 0)

    pltpu.sync_copy(srcv_h.at[wid], src_v)
    pltpu.sync_copy(dstv_h.at[wid], dst_v)

    def zero_acc():
      def zcp(i, _):
        pltpu.sync_copy(zbuf, acc.at[pl.ds(r0 + i * ZR, ZR)])
        return 0
      lax.fori_loop(0, RPT // ZR, zcp, 0)

    for p in range(nch):
      zero_acc()
      plsc.subcore_barrier()

      base = p * NP

      def blk(j, _):
        for k in range(B // L):
          idxb[pl.ds(k * L, L)] = src_v[j, pl.ds(k * L, L)] + base
        pltpu.async_copy(table_h.at[idxb], g, sem).wait()
        pltpu.sync_copy(ubv_h.at[wid, j], ubb)

        def row(i, _):
          uv = ubb[i, :]
          for k in range(DC // L):
            g[i, pl.ds(DC + k * L, L)] = uv * g[i, pl.ds(DC + k * L, L)]
          return 0
        lax.fori_loop(0, B, row, 0)

        pltpu.sync_copy(g, acc.at[dst_v.at[j]], add=True)
        return 0
      lax.fori_loop(0, nbk, blk, 0)
      plsc.subcore_barrier()

      pltpu.sync_copy(acc.at[pl.ds(r0, RPT)],
                      sa_out.at[cid, p, pl.ds(r0, RPT)])
      plsc.subcore_barrier()

    if with_cnt:
      zero_acc()
      ov = jnp.full((L,), 1.0, jnp.float32)

      def orow(i, _):
        for k in range(W2C // L):
          g[i, pl.ds(k * L, L)] = ov
        return 0
      lax.fori_loop(0, B, orow, 0)
      plsc.subcore_barrier()

      def cblk(j, _):
        pltpu.sync_copy(g, acc.at[dst_v.at[j]], add=True)
        return 0
      lax.fori_loop(0, nbk, cblk, 0)
      plsc.subcore_barrier()
      pltpu.sync_copy(acc.at[pl.ds(r0, RPT)], cnt_out.at[cid, pl.ds(r0, RPT)])

  return body


def _make_tc_layer(nch, d, chunked_out):
  R = 1024
  grid = (NP // R,)

  def body(sa_ref, c_ref, x_ref, wcat_ref, wrcat_ref, b_ref, o_ref):
    cnt = jnp.sum(c_ref[0] + c_ref[1], axis=1) * (1.0 / W2C)
    inv = 1.0 / jnp.maximum(cnt, 1.0)
    conv = jnp.zeros((R, d), jnp.float32)
    root = jnp.zeros((R, d), jnp.float32)
    for c in range(nch):
      sa = sa_ref[0, c] + sa_ref[1, c]
      conv += jnp.dot(sa, wcat_ref[c], preferred_element_type=jnp.float32,
                      precision="highest")
      root += jnp.dot(x_ref[c], wrcat_ref[c], preferred_element_type=jnp.float32,
                      precision="highest")
    res = conv * inv[:, None] + root + b_ref[...]
    if chunked_out:
      for c in range(nch):
        o_ref[c, :, 0:DC] = res[:, c * DC:(c + 1) * DC]
        o_ref[c, :, DC:W2C] = res[:, c * DC:(c + 1) * DC]
    else:
      o_ref[...] = res

  if chunked_out:
    out_shape = jax.ShapeDtypeStruct((nch, NP, W2C), jnp.float32)
    out_spec = pl.BlockSpec((nch, R, W2C), lambda i: (0, i, 0))
  else:
    out_shape = jax.ShapeDtypeStruct((NP, d), jnp.float32)
    out_spec = pl.BlockSpec((R, d), lambda i: (i, 0))

  return pl.pallas_call(
      body,
      grid=grid,
      in_specs=[
          pl.BlockSpec((NC, nch, R, W2C), lambda i: (0, 0, i, 0)),
          pl.BlockSpec((NC, R, W2C), lambda i: (0, i, 0)),
          pl.BlockSpec((nch, R, W2C), lambda i: (0, i, 0)),
          pl.BlockSpec((nch, W2C, d), lambda i: (0, 0, 0)),
          pl.BlockSpec((nch, W2C, d), lambda i: (0, 0, 0)),
          pl.BlockSpec((1, d), lambda i: (0, 0)),
      ],
      out_specs=out_spec,
      out_shape=out_shape,
  )


def kernel(t, x, edge_index, edge_attr, W1, Wr1, b1, W2, Wr2, b2):
  del t
  n, d = x.shape
  e = edge_index.shape[1]
  nch = d // DC
  nbk = -(-e // (NW * B))
  epad = NW * nbk * B

  u = jnp.clip(edge_attr[:, 0], 0.0, 1.0)
  pad = epad - e
  src_p = jnp.concatenate([edge_index[0], jnp.zeros((pad,), jnp.int32)])
  dst_p = jnp.concatenate([edge_index[1], jnp.full((pad,), NP - 1, jnp.int32)])
  u_p = jnp.concatenate([u, jnp.zeros((pad,), jnp.float32)])
  srcv = src_p.reshape(NW, nbk, B)
  dstv = dst_p.reshape(NW, nbk, B)
  ubv = jnp.broadcast_to(u_p.reshape(NW, nbk, B, 1), (NW, nbk, B, L))

  xr = x.reshape(n, nch, DC).transpose(1, 0, 2)
  xrp = jnp.pad(xr, ((0, 0), (0, NP - n), (0, 0)))
  tab1 = jnp.concatenate([xrp, xrp], axis=-1)

  def wcat(w0, w1):
    return jnp.concatenate([w0.reshape(nch, DC, d),
                            (w1 - w0).reshape(nch, DC, d)], axis=1)

  def wrcat(wr):
    wrr = wr.reshape(nch, DC, d)
    return jnp.concatenate([wrr, jnp.zeros_like(wrr)], axis=1)

  sc1 = _make_sc_pass(nbk, nch, with_cnt=True)
  sc2 = _make_sc_pass(nbk, nch, with_cnt=False)
  tc1 = _make_tc_layer(nch, d, chunked_out=True)
  tc2 = _make_tc_layer(nch, d, chunked_out=False)

  sa1, cnt = sc1(tab1.reshape(nch * NP, W2C), srcv, dstv, ubv)
  hr = tc1(sa1, cnt, tab1, wcat(W1[0], W1[1]), wrcat(Wr1), b1.reshape(1, d))
  (sa2,) = sc2(hr.reshape(nch * NP, W2C), srcv, dstv, ubv)
  out = tc2(sa2, cnt, hr, wcat(W2[0], W2[1]), wrcat(Wr2), b2.reshape(1, d))
  return out[:n]


# speedup vs baseline: 1.0762x; 1.0762x over previous
import functools

import jax
import jax.numpy as jnp
from jax import lax
from jax.experimental import pallas as pl
from jax.experimental.pallas import tpu as pltpu
from jax.experimental.pallas import tpu_sc as plsc

NC = 2
NS = 16
L = 16
NW = NC * NS

B = 128
DC = 64
W2C = 2 * DC
NP = 10112
RPT = NP // NS


def _make_sc_pass(nbk, nch, with_cnt):
  mesh = plsc.VectorSubcoreMesh(core_axis_name="c", subcore_axis_name="s",
                                num_cores=NC, num_subcores=NS)
  out_type = [jax.ShapeDtypeStruct((NC, nch, NP, W2C), jnp.float32)]
  scratch = [
      pltpu.VMEM((nbk, B), jnp.int32),
      pltpu.VMEM((nbk, B), jnp.int32),
      pltpu.VMEM((B,), jnp.int32),
      pltpu.VMEM((2, B, W2C), jnp.float32),
      pltpu.VMEM((B * L // 128, 128), jnp.float32),
      pltpu.VMEM_SHARED((NP, W2C), jnp.float32),
      pltpu.SemaphoreType.DMA,
      pltpu.SemaphoreType.DMA,
      pltpu.SemaphoreType.DMA,
      pltpu.SemaphoreType.DMA,
      pltpu.SemaphoreType.DMA,
      pltpu.SemaphoreType.DMA,
  ]
  if with_cnt:
    out_type.append(jax.ShapeDtypeStruct((NC, NP, W2C), jnp.float32))

  @functools.partial(pl.kernel, out_type=tuple(out_type), mesh=mesh,
                     scratch_types=scratch)
  def body(table_h, srcv_h, dstv_h, ubv_h, zrs_h, *rest):
    if with_cnt:
      (sa_out, cnt_out, src_v, dst_v, idx, g, ub,
       acc, gs0, gs1, us0, us1, ss0, ss1) = rest
    else:
      (sa_out, src_v, dst_v, idx, g, ub,
       acc, gs0, gs1, us0, us1, ss0, ss1) = rest
    gsem, usem, ssem = (gs0, gs1), (us0, us1), (ss0, ss1)
    cid = lax.axis_index("c")
    sid = lax.axis_index("s")
    wid = sid * NC + cid
    r0 = sid * RPT

    pltpu.sync_copy(srcv_h.at[wid], src_v)
    pltpu.sync_copy(dstv_h.at[wid], dst_v)

    def zero_acc():
      pltpu.sync_copy(zrs_h, acc.at[pl.ds(r0, RPT)])

    def comp_idx(j, base):
      for k in range(B // L):
        idx[pl.ds(k * L, L)] = src_v[j, pl.ds(k * L, L)] + base

    nsub = 128 // L

    def scale_hi(par):
      def row8(i8, _):
        for sub in range(nsub):
          uv = ub[i8, pl.ds(sub * L, L)]
          i = i8 * nsub + sub
          for k in range(DC // L):
            g[par, i, pl.ds(DC + k * L, L)] = (
                uv * g[par, i, pl.ds(DC + k * L, L)])
        return 0
      lax.fori_loop(0, B * L // 128, row8, 0)

    for p in range(nch):
      zero_acc()
      plsc.subcore_barrier()

      base = p * NP

      def pair(jj, _):
        j0 = jj * 2
        comp_idx(j0, base)
        dg0 = pltpu.async_copy(table_h.at[idx], g.at[0], gsem[0])
        du0 = pltpu.async_copy(ubv_h.at[wid, j0], ub, usem[0])
        dg0.wait()
        comp_idx(j0 + 1, base)
        dg1 = pltpu.async_copy(table_h.at[idx], g.at[1], gsem[1])
        du0.wait()
        scale_hi(0)
        du1 = pltpu.async_copy(ubv_h.at[wid, j0 + 1], ub, usem[1])
        ds0 = pltpu.async_copy(g.at[0], acc.at[dst_v.at[j0]], ssem[0],
                               add=True)
        dg1.wait()
        du1.wait()
        scale_hi(1)
        ds0.wait()
        ds1 = pltpu.async_copy(g.at[1], acc.at[dst_v.at[j0 + 1]], ssem[1],
                               add=True)
        ds1.wait()
        return 0
      lax.fori_loop(0, nbk // 2, pair, 0)
      plsc.subcore_barrier()

      pltpu.sync_copy(acc.at[pl.ds(r0, RPT)],
                      sa_out.at[cid, p, pl.ds(r0, RPT)])
      plsc.subcore_barrier()

    if with_cnt:
      zero_acc()
      ov = jnp.full((L,), 1.0, jnp.float32)

      def orow(i, _):
        for k in range(W2C // L):
          g[0, i, pl.ds(k * L, L)] = ov
        return 0
      lax.fori_loop(0, B, orow, 0)
      plsc.subcore_barrier()

      def cpair(jj, _):
        d0 = pltpu.async_copy(g.at[0], acc.at[dst_v.at[jj * 2]], ssem[0],
                              add=True)
        d0.wait()
        d1 = pltpu.async_copy(g.at[0], acc.at[dst_v.at[jj * 2 + 1]], ssem[1],
                              add=True)
        d1.wait()
        return 0
      lax.fori_loop(0, nbk // 2, cpair, 0)
      plsc.subcore_barrier()
      pltpu.sync_copy(acc.at[pl.ds(r0, RPT)], cnt_out.at[cid, pl.ds(r0, RPT)])

  return body


def _make_tc_layer(nch, d, chunked_out):
  R = 1264
  grid = (NP // R,)

  def body(sa_ref, c_ref, x_ref, wcat_ref, wrcat_ref, b_ref, o_ref):
    cnt = jnp.sum(c_ref[0] + c_ref[1], axis=1) * (1.0 / W2C)
    inv = 1.0 / jnp.maximum(cnt, 1.0)
    conv = jnp.zeros((R, d), jnp.float32)
    root = jnp.zeros((R, d), jnp.float32)
    for c in range(nch):
      sa = sa_ref[0, c] + sa_ref[1, c]
      conv += jnp.dot(sa, wcat_ref[c], preferred_element_type=jnp.float32,
                      precision="highest")
      root += jnp.dot(x_ref[c], wrcat_ref[c], preferred_element_type=jnp.float32,
                      precision="highest")
    res = conv * inv[:, None] + root + b_ref[...]
    if chunked_out:
      for c in range(nch):
        o_ref[c, :, 0:DC] = res[:, c * DC:(c + 1) * DC]
        o_ref[c, :, DC:W2C] = res[:, c * DC:(c + 1) * DC]
    else:
      o_ref[...] = res

  if chunked_out:
    out_shape = jax.ShapeDtypeStruct((nch, NP, W2C), jnp.float32)
    out_spec = pl.BlockSpec((nch, R, W2C), lambda i: (0, i, 0))
  else:
    out_shape = jax.ShapeDtypeStruct((NP, d), jnp.float32)
    out_spec = pl.BlockSpec((R, d), lambda i: (i, 0))

  return pl.pallas_call(
      body,
      grid=grid,
      in_specs=[
          pl.BlockSpec((NC, nch, R, W2C), lambda i: (0, 0, i, 0)),
          pl.BlockSpec((NC, R, W2C), lambda i: (0, i, 0)),
          pl.BlockSpec((nch, R, W2C), lambda i: (0, i, 0)),
          pl.BlockSpec((nch, W2C, d), lambda i: (0, 0, 0)),
          pl.BlockSpec((nch, W2C, d), lambda i: (0, 0, 0)),
          pl.BlockSpec((1, d), lambda i: (0, 0)),
      ],
      out_specs=out_spec,
      out_shape=out_shape,
  )


def kernel(t, x, edge_index, edge_attr, W1, Wr1, b1, W2, Wr2, b2):
  del t
  n, d = x.shape
  e = edge_index.shape[1]
  nch = d // DC
  nbk = -(-e // (NW * B))
  epad = NW * nbk * B

  u = jnp.clip(edge_attr[:, 0], 0.0, 1.0)
  pad = epad - e
  src_p = jnp.concatenate([edge_index[0], jnp.zeros((pad,), jnp.int32)])
  dst_p = jnp.concatenate([edge_index[1], jnp.full((pad,), NP - 1, jnp.int32)])
  u_p = jnp.concatenate([u, jnp.zeros((pad,), jnp.float32)])
  srcv = src_p.reshape(NW, nbk, B)
  dstv = dst_p.reshape(NW, nbk, B)
  ubv = jnp.broadcast_to(u_p.reshape(NW, nbk, B, 1),
                         (NW, nbk, B, L)).reshape(NW, nbk, B * L // 128, 128)

  xr = x.reshape(n, nch, DC).transpose(1, 0, 2)
  xrp = jnp.pad(xr, ((0, 0), (0, NP - n), (0, 0)))
  tab1 = jnp.concatenate([xrp, xrp], axis=-1)

  def wcat(w0, w1):
    return jnp.concatenate([w0.reshape(nch, DC, d),
                            (w1 - w0).reshape(nch, DC, d)], axis=1)

  def wrcat(wr):
    wrr = wr.reshape(nch, DC, d)
    return jnp.concatenate([wrr, jnp.zeros_like(wrr)], axis=1)

  sc1 = _make_sc_pass(nbk, nch, with_cnt=True)
  sc2 = _make_sc_pass(nbk, nch, with_cnt=False)
  tc1 = _make_tc_layer(nch, d, chunked_out=True)
  tc2 = _make_tc_layer(nch, d, chunked_out=False)

  zrs = jnp.zeros((RPT, W2C), jnp.float32)
  sa1, cnt = sc1(tab1.reshape(nch * NP, W2C), srcv, dstv, ubv, zrs)
  hr = tc1(sa1, cnt, tab1, wcat(W1[0], W1[1]), wrcat(Wr1), b1.reshape(1, d))
  (sa2,) = sc2(hr.reshape(nch * NP, W2C), srcv, dstv, ubv, zrs)
  out = tc2(sa2, cnt, hr, wcat(W2[0], W2[1]), wrcat(Wr2), b2.reshape(1, d))
  return out[:n]


# speedup vs baseline: 1.3556x; 1.2596x over previous
import functools

import jax
import jax.numpy as jnp
from jax import lax
from jax.experimental import pallas as pl
from jax.experimental.pallas import tpu as pltpu
from jax.experimental.pallas import tpu_sc as plsc

NC = 2
NS = 16
L = 16
NW = NC * NS

B = 128
DC = 64
W2C = 2 * DC
NP = 10112
RPT = NP // NS
NUB = B * L // 128

SBB = 16
SB0 = 4
SB1 = 1
SBT = SB0 + SB1


def _make_sc_pass(nch, with_cnt):
  mesh = plsc.VectorSubcoreMesh(core_axis_name="c", subcore_axis_name="s",
                                num_cores=NC, num_subcores=NS)
  out_type = [jax.ShapeDtypeStruct((NC, nch, NP, W2C), jnp.float32)]
  scratch = [
      pltpu.VMEM((2 * SBB, B), jnp.int32),
      pltpu.VMEM((2, B, W2C), jnp.float32),
      pltpu.VMEM((NUB, 128), jnp.float32),
      pltpu.VMEM_SHARED((NP, W2C), jnp.float32),
      pltpu.SemaphoreType.DMA,
      pltpu.SemaphoreType.DMA,
      pltpu.SemaphoreType.DMA,
      pltpu.SemaphoreType.DMA,
      pltpu.SemaphoreType.DMA,
      pltpu.SemaphoreType.DMA,
  ]
  if with_cnt:
    out_type.append(jax.ShapeDtypeStruct((NC, NP, W2C), jnp.float32))

  @functools.partial(pl.kernel, out_type=tuple(out_type), mesh=mesh,
                     scratch_types=scratch)
  def body(table_h, edata_h, ubv_h, zrs_h, *rest):
    if with_cnt:
      sa_out, cnt_out, ebuf, g, ub, acc, gs0, gs1, us0, us1, ss0, ss1 = rest
    else:
      sa_out, ebuf, g, ub, acc, gs0, gs1, us0, us1, ss0, ss1 = rest
    gsem, usem, ssem = (gs0, gs1), (us0, us1), (ss0, ss1)
    cid = lax.axis_index("c")
    sid = lax.axis_index("s")
    r0 = sid * RPT
    roff = cid * SB0
    nsb = jnp.where(cid == 0, SB0, SB1)

    def zero_acc():
      pltpu.sync_copy(zrs_h, acc.at[pl.ds(r0, RPT)])

    def scale_hi(par):
      nspl = 128 // L

      def row8(i8, _):
        for sub in range(nspl):
          uv = ub[i8, pl.ds(sub * L, L)]
          i = i8 * nspl + sub
          for k in range(DC // L):
            g[par, i, pl.ds(DC + k * L, L)] = (
                uv * g[par, i, pl.ds(DC + k * L, L)])
        return 0
      lax.fori_loop(0, NUB, row8, 0)

    def pass_body(p, _):
      zero_acc()
      plsc.subcore_barrier()

      def sblk(s, _):
        pltpu.sync_copy(edata_h.at[p, sid, roff + s], ebuf)
        ds_prev = None
        for l in range(SBB // 2):
          dg0 = pltpu.async_copy(table_h.at[ebuf.at[4 * l]], g.at[0],
                                 gsem[0])
          du0 = pltpu.async_copy(ubv_h.at[sid, roff + s, 2 * l], ub, usem[0])
          dg0.wait()
          du0.wait()
          if ds_prev is not None:
            ds_prev.wait()
          dg1 = pltpu.async_copy(table_h.at[ebuf.at[4 * l + 2]], g.at[1],
                                 gsem[1])
          scale_hi(0)
          du1 = pltpu.async_copy(ubv_h.at[sid, roff + s, 2 * l + 1], ub,
                                 usem[1])
          ds0 = pltpu.async_copy(g.at[0], acc.at[ebuf.at[4 * l + 1]],
                                 ssem[0], add=True)
          dg1.wait()
          du1.wait()
          scale_hi(1)
          ds0.wait()
          ds_prev = pltpu.async_copy(g.at[1], acc.at[ebuf.at[4 * l + 3]],
                                     ssem[1], add=True)
        ds_prev.wait()
        return 0
      lax.fori_loop(0, nsb, sblk, 0)
      plsc.subcore_barrier()

      pltpu.sync_copy(acc.at[pl.ds(r0, RPT)],
                      sa_out.at[cid, p, pl.ds(r0, RPT)])
      plsc.subcore_barrier()
      return 0
    lax.fori_loop(0, nch, pass_body, 0)

    if with_cnt:
      zero_acc()
      ov = jnp.full((L,), 1.0, jnp.float32)

      def orow(i, _):
        for k in range(W2C // L):
          g[0, i, pl.ds(k * L, L)] = ov
        return 0
      lax.fori_loop(0, B, orow, 0)
      plsc.subcore_barrier()

      def csblk(s, _):
        pltpu.sync_copy(edata_h.at[0, sid, roff + s], ebuf)
        d_prev = None
        for b in range(SBB):
          d = pltpu.async_copy(g.at[0], acc.at[ebuf.at[2 * b + 1]],
                               ssem[b % 2], add=True)
          if d_prev is not None:
            d_prev.wait()
          d_prev = d
        d_prev.wait()
        return 0
      lax.fori_loop(0, nsb, csblk, 0)
      plsc.subcore_barrier()
      pltpu.sync_copy(acc.at[pl.ds(r0, RPT)], cnt_out.at[cid, pl.ds(r0, RPT)])

  return body


def _make_tc_layer(nch, d, chunked_out):
  R = 1264
  grid = (NP // R,)

  def body(sa_ref, c_ref, x_ref, wcat_ref, wrcat_ref, b_ref, o_ref):
    cnt = jnp.sum(c_ref[0] + c_ref[1], axis=1) * (1.0 / W2C)
    inv = 1.0 / jnp.maximum(cnt, 1.0)
    conv = jnp.zeros((R, d), jnp.float32)
    root = jnp.zeros((R, d), jnp.float32)
    for c in range(nch):
      sa = sa_ref[0, c] + sa_ref[1, c]
      conv += jnp.dot(sa, wcat_ref[c], preferred_element_type=jnp.float32,
                      precision="highest")
      root += jnp.dot(x_ref[c], wrcat_ref[c], preferred_element_type=jnp.float32,
                      precision="highest")
    res = conv * inv[:, None] + root + b_ref[...]
    if chunked_out:
      for c in range(nch):
        o_ref[c, :, 0:DC] = res[:, c * DC:(c + 1) * DC]
        o_ref[c, :, DC:W2C] = res[:, c * DC:(c + 1) * DC]
    else:
      o_ref[...] = res

  if chunked_out:
    out_shape = jax.ShapeDtypeStruct((nch, NP, W2C), jnp.float32)
    out_spec = pl.BlockSpec((nch, R, W2C), lambda i: (0, i, 0))
  else:
    out_shape = jax.ShapeDtypeStruct((NP, d), jnp.float32)
    out_spec = pl.BlockSpec((R, d), lambda i: (i, 0))

  return pl.pallas_call(
      body,
      grid=grid,
      in_specs=[
          pl.BlockSpec((NC, nch, R, W2C), lambda i: (0, 0, i, 0)),
          pl.BlockSpec((NC, R, W2C), lambda i: (0, i, 0)),
          pl.BlockSpec((nch, R, W2C), lambda i: (0, i, 0)),
          pl.BlockSpec((nch, W2C, d), lambda i: (0, 0, 0)),
          pl.BlockSpec((nch, W2C, d), lambda i: (0, 0, 0)),
          pl.BlockSpec((1, d), lambda i: (0, 0)),
      ],
      out_specs=out_spec,
      out_shape=out_shape,
  )


def kernel(t, x, edge_index, edge_attr, W1, Wr1, b1, W2, Wr2, b2):
  del t
  n, d = x.shape
  e = edge_index.shape[1]
  nch = d // DC
  epad = NS * SBT * SBB * B

  u = jnp.clip(edge_attr[:, 0], 0.0, 1.0)
  pad = epad - e
  src_p = jnp.concatenate([edge_index[0], jnp.zeros((pad,), jnp.int32)])
  dst_p = jnp.concatenate([edge_index[1], jnp.full((pad,), NP - 1, jnp.int32)])
  u_p = jnp.concatenate([u, jnp.zeros((pad,), jnp.float32)])
  s_r = src_p.reshape(NS, SBT, SBB, B)
  d_r = dst_p.reshape(NS, SBT, SBB, B)
  edata = jnp.stack(
      [jnp.stack([s_r + p * NP, d_r], axis=3) for p in range(nch)],
      axis=0).reshape(nch, NS, SBT, 2 * SBB, B)
  ubv = jnp.broadcast_to(u_p.reshape(NS, SBT, SBB, B, 1),
                         (NS, SBT, SBB, B, L)).reshape(NS, SBT, SBB, NUB, 128)

  xr = x.reshape(n, nch, DC).transpose(1, 0, 2)
  xrp = jnp.pad(xr, ((0, 0), (0, NP - n), (0, 0)))
  tab1 = jnp.concatenate([xrp, xrp], axis=-1)

  def wcat(w0, w1):
    return jnp.concatenate([w0.reshape(nch, DC, d),
                            (w1 - w0).reshape(nch, DC, d)], axis=1)

  def wrcat(wr):
    wrr = wr.reshape(nch, DC, d)
    return jnp.concatenate([wrr, jnp.zeros_like(wrr)], axis=1)

  sc1 = _make_sc_pass(nch, with_cnt=True)
  sc2 = _make_sc_pass(nch, with_cnt=False)
  tc1 = _make_tc_layer(nch, d, chunked_out=True)
  tc2 = _make_tc_layer(nch, d, chunked_out=False)

  zrs = jnp.zeros((RPT, W2C), jnp.float32)
  sa1, cnt = sc1(tab1.reshape(nch * NP, W2C), edata, ubv, zrs)
  hr = tc1(sa1, cnt, tab1, wcat(W1[0], W1[1]), wrcat(Wr1), b1.reshape(1, d))
  (sa2,) = sc2(hr.reshape(nch * NP, W2C), edata, ubv, zrs)
  out = tc2(sa2, cnt, hr, wcat(W2[0], W2[1]), wrcat(Wr2), b2.reshape(1, d))
  return out[:n]


# speedup vs baseline: 1.3558x; 1.0001x over previous
import functools

import jax
import jax.numpy as jnp
from jax import lax
from jax.experimental import pallas as pl
from jax.experimental.pallas import tpu as pltpu
from jax.experimental.pallas import tpu_sc as plsc

NC = 2
NS = 16
L = 16
NW = NC * NS

B = 128
DC = 64
W2C = 2 * DC
NP = 10112
RPT = NP // NS
NUB = B * L // 128

SBB = 16
SB0 = 4
SB1 = 1
SBT = SB0 + SB1


def _make_sc_pass(nch, with_cnt):
  mesh = plsc.VectorSubcoreMesh(core_axis_name="c", subcore_axis_name="s",
                                num_cores=NC, num_subcores=NS)
  out_type = [jax.ShapeDtypeStruct((NC, nch, NP, W2C), jnp.float32)]
  scratch = [
      pltpu.VMEM((2 * SBB, B), jnp.int32),
      pltpu.VMEM((2, B, W2C), jnp.float32),
      pltpu.VMEM((2, NUB, 128), jnp.float32),
      pltpu.VMEM_SHARED((NP, W2C), jnp.float32),
      pltpu.SemaphoreType.DMA,
      pltpu.SemaphoreType.DMA,
      pltpu.SemaphoreType.DMA,
      pltpu.SemaphoreType.DMA,
      pltpu.SemaphoreType.DMA,
      pltpu.SemaphoreType.DMA,
  ]
  if with_cnt:
    out_type.append(jax.ShapeDtypeStruct((NC, NP, W2C), jnp.float32))

  @functools.partial(pl.kernel, out_type=tuple(out_type), mesh=mesh,
                     scratch_types=scratch)
  def body(table_h, edata_h, ubv_h, zrs_h, *rest):
    if with_cnt:
      sa_out, cnt_out, ebuf, g, ub, acc, gs0, gs1, us0, us1, ss0, ss1 = rest
    else:
      sa_out, ebuf, g, ub, acc, gs0, gs1, us0, us1, ss0, ss1 = rest
    gsem, usem, ssem = (gs0, gs1), (us0, us1), (ss0, ss1)
    cid = lax.axis_index("c")
    sid = lax.axis_index("s")
    r0 = sid * RPT
    roff = cid * SB0
    nsb = jnp.where(cid == 0, SB0, SB1)

    def zero_acc():
      pltpu.sync_copy(zrs_h, acc.at[pl.ds(r0, RPT)])

    def scale_hi(par):
      nspl = 128 // L

      def row8(i8, _):
        for sub in range(nspl):
          uv = ub[par, i8, pl.ds(sub * L, L)]
          i = i8 * nspl + sub
          for k in range(DC // L):
            g[par, i, pl.ds(DC + k * L, L)] = (
                uv * g[par, i, pl.ds(DC + k * L, L)])
        return 0
      lax.fori_loop(0, NUB, row8, 0)

    def pass_body(p, _):
      zero_acc()
      plsc.subcore_barrier()

      def sblk(s, _):
        pltpu.sync_copy(edata_h.at[p, sid, roff + s], ebuf)
        npair = SBB // 2
        dgn, dun = [None] * npair, [None] * npair

        def issue0(l):
          dgn[l] = pltpu.async_copy(table_h.at[ebuf.at[4 * l]], g.at[0],
                                    gsem[0])
          dun[l] = pltpu.async_copy(ubv_h.at[sid, roff + s, 2 * l],
                                    ub.at[0], usem[0])
        issue0(0)
        ds_prev = None
        for l in range(npair):
          dgn[l].wait()
          dun[l].wait()
          if ds_prev is not None:
            ds_prev.wait()
          dg1 = pltpu.async_copy(table_h.at[ebuf.at[4 * l + 2]], g.at[1],
                                 gsem[1])
          scale_hi(0)
          du1 = pltpu.async_copy(ubv_h.at[sid, roff + s, 2 * l + 1],
                                 ub.at[1], usem[1])
          ds0 = pltpu.async_copy(g.at[0], acc.at[ebuf.at[4 * l + 1]],
                                 ssem[0], add=True)
          dg1.wait()
          du1.wait()
          scale_hi(1)
          ds0.wait()
          ds_prev = pltpu.async_copy(g.at[1], acc.at[ebuf.at[4 * l + 3]],
                                     ssem[1], add=True)
          if l + 1 < npair:
            issue0(l + 1)
        ds_prev.wait()
        return 0
      lax.fori_loop(0, nsb, sblk, 0)
      plsc.subcore_barrier()

      pltpu.sync_copy(acc.at[pl.ds(r0, RPT)],
                      sa_out.at[cid, p, pl.ds(r0, RPT)])
      plsc.subcore_barrier()
      return 0
    lax.fori_loop(0, nch, pass_body, 0)

    if with_cnt:
      zero_acc()
      ov = jnp.full((L,), 1.0, jnp.float32)

      def orow(i, _):
        for k in range(W2C // L):
          g[0, i, pl.ds(k * L, L)] = ov
        return 0
      lax.fori_loop(0, B, orow, 0)
      plsc.subcore_barrier()

      def csblk(s, _):
        pltpu.sync_copy(edata_h.at[0, sid, roff + s], ebuf)
        d_prev = None
        for b in range(SBB):
          d = pltpu.async_copy(g.at[0], acc.at[ebuf.at[2 * b + 1]],
                               ssem[b % 2], add=True)
          if d_prev is not None:
            d_prev.wait()
          d_prev = d
        d_prev.wait()
        return 0
      lax.fori_loop(0, nsb, csblk, 0)
      plsc.subcore_barrier()
      pltpu.sync_copy(acc.at[pl.ds(r0, RPT)], cnt_out.at[cid, pl.ds(r0, RPT)])

  return body


def _make_tc_layer(nch, d, chunked_out):
  R = 1264
  grid = (NP // R,)

  def body(sa_ref, c_ref, x_ref, wcat_ref, wrcat_ref, b_ref, o_ref):
    cnt = jnp.sum(c_ref[0] + c_ref[1], axis=1) * (1.0 / W2C)
    inv = 1.0 / jnp.maximum(cnt, 1.0)
    conv = jnp.zeros((R, d), jnp.float32)
    root = jnp.zeros((R, d), jnp.float32)
    for c in range(nch):
      sa = sa_ref[0, c] + sa_ref[1, c]
      conv += jnp.dot(sa, wcat_ref[c], preferred_element_type=jnp.float32,
                      precision="highest")
      root += jnp.dot(x_ref[c], wrcat_ref[c], preferred_element_type=jnp.float32,
                      precision="highest")
    res = conv * inv[:, None] + root + b_ref[...]
    if chunked_out:
      for c in range(nch):
        o_ref[c, :, 0:DC] = res[:, c * DC:(c + 1) * DC]
        o_ref[c, :, DC:W2C] = res[:, c * DC:(c + 1) * DC]
    else:
      o_ref[...] = res

  if chunked_out:
    out_shape = jax.ShapeDtypeStruct((nch, NP, W2C), jnp.float32)
    out_spec = pl.BlockSpec((nch, R, W2C), lambda i: (0, i, 0))
  else:
    out_shape = jax.ShapeDtypeStruct((NP, d), jnp.float32)
    out_spec = pl.BlockSpec((R, d), lambda i: (i, 0))

  return pl.pallas_call(
      body,
      grid=grid,
      in_specs=[
          pl.BlockSpec((NC, nch, R, W2C), lambda i: (0, 0, i, 0)),
          pl.BlockSpec((NC, R, W2C), lambda i: (0, i, 0)),
          pl.BlockSpec((nch, R, W2C), lambda i: (0, i, 0)),
          pl.BlockSpec((nch, W2C, d), lambda i: (0, 0, 0)),
          pl.BlockSpec((nch, W2C, d), lambda i: (0, 0, 0)),
          pl.BlockSpec((1, d), lambda i: (0, 0)),
      ],
      out_specs=out_spec,
      out_shape=out_shape,
  )


def kernel(t, x, edge_index, edge_attr, W1, Wr1, b1, W2, Wr2, b2):
  del t
  n, d = x.shape
  e = edge_index.shape[1]
  nch = d // DC
  epad = NS * SBT * SBB * B

  u = jnp.clip(edge_attr[:, 0], 0.0, 1.0)
  pad = epad - e
  src_p = jnp.concatenate([edge_index[0], jnp.zeros((pad,), jnp.int32)])
  dst_p = jnp.concatenate([edge_index[1], jnp.full((pad,), NP - 1, jnp.int32)])
  u_p = jnp.concatenate([u, jnp.zeros((pad,), jnp.float32)])
  s_r = src_p.reshape(NS, SBT, SBB, B)
  d_r = dst_p.reshape(NS, SBT, SBB, B)
  edata = jnp.stack(
      [jnp.stack([s_r + p * NP, d_r], axis=3) for p in range(nch)],
      axis=0).reshape(nch, NS, SBT, 2 * SBB, B)
  ubv = jnp.broadcast_to(u_p.reshape(NS, SBT, SBB, B, 1),
                         (NS, SBT, SBB, B, L)).reshape(NS, SBT, SBB, NUB, 128)

  xr = x.reshape(n, nch, DC).transpose(1, 0, 2)
  xrp = jnp.pad(xr, ((0, 0), (0, NP - n), (0, 0)))
  tab1 = jnp.concatenate([xrp, xrp], axis=-1)

  def wcat(w0, w1):
    return jnp.concatenate([w0.reshape(nch, DC, d),
                            (w1 - w0).reshape(nch, DC, d)], axis=1)

  def wrcat(wr):
    wrr = wr.reshape(nch, DC, d)
    return jnp.concatenate([wrr, jnp.zeros_like(wrr)], axis=1)

  sc1 = _make_sc_pass(nch, with_cnt=True)
  sc2 = _make_sc_pass(nch, with_cnt=False)
  tc1 = _make_tc_layer(nch, d, chunked_out=True)
  tc2 = _make_tc_layer(nch, d, chunked_out=False)

  zrs = jnp.zeros((RPT, W2C), jnp.float32)
  sa1, cnt = sc1(tab1.reshape(nch * NP, W2C), edata, ubv, zrs)
  hr = tc1(sa1, cnt, tab1, wcat(W1[0], W1[1]), wrcat(Wr1), b1.reshape(1, d))
  (sa2,) = sc2(hr.reshape(nch * NP, W2C), edata, ubv, zrs)
  out = tc2(sa2, cnt, hr, wcat(W2[0], W2[1]), wrcat(Wr2), b2.reshape(1, d))
  return out[:n]


# speedup vs baseline: 1.3559x; 1.0001x over previous
import functools

import jax
import jax.numpy as jnp
from jax import lax
from jax.experimental import pallas as pl
from jax.experimental.pallas import tpu as pltpu
from jax.experimental.pallas import tpu_sc as plsc

NC = 2
NS = 16
L = 16
NW = NC * NS

B = 128
DC = 64
W2C = 2 * DC
NP = 10112
RPT = NP // NS
NUB = B * L // 128

SBB = 16
SB0 = 4
SB1 = 1
SBT = SB0 + SB1


def _make_sc_pass(nch, with_cnt):
  mesh = plsc.VectorSubcoreMesh(core_axis_name="c", subcore_axis_name="s",
                                num_cores=NC, num_subcores=NS)
  out_type = [jax.ShapeDtypeStruct((NC, nch, NP, W2C), jnp.float32)]
  scratch = [
      pltpu.VMEM((2 * SBB, B), jnp.int32),
      pltpu.VMEM((2, B, W2C), jnp.float32),
      pltpu.VMEM((2, NUB, 128), jnp.float32),
      pltpu.VMEM_SHARED((NP, W2C), jnp.float32),
      pltpu.SemaphoreType.DMA,
      pltpu.SemaphoreType.DMA,
      pltpu.SemaphoreType.DMA,
      pltpu.SemaphoreType.DMA,
      pltpu.SemaphoreType.DMA,
      pltpu.SemaphoreType.DMA,
  ]
  if with_cnt:
    out_type.append(jax.ShapeDtypeStruct((NC, NP, W2C), jnp.float32))

  @functools.partial(pl.kernel, out_type=tuple(out_type), mesh=mesh,
                     scratch_types=scratch)
  def body(table_h, edata_h, ubv_h, zrs_h, *rest):
    if with_cnt:
      sa_out, cnt_out, ebuf, g, ub, acc, gs0, gs1, us0, us1, ss0, ss1 = rest
    else:
      sa_out, ebuf, g, ub, acc, gs0, gs1, us0, us1, ss0, ss1 = rest
    gsem, usem, ssem = (gs0, gs1), (us0, us1), (ss0, ss1)
    cid = lax.axis_index("c")
    sid = lax.axis_index("s")
    r0 = sid * RPT
    roff = cid * SB0
    nsb = jnp.where(cid == 0, SB0, SB1)

    def zero_acc():
      pltpu.sync_copy(zrs_h, acc.at[pl.ds(r0, RPT)])

    def scale_hi(par):
      nspl = 128 // L

      def row8(i8, _):
        for sub in range(nspl):
          uv = ub[par, i8, pl.ds(sub * L, L)]
          i = i8 * nspl + sub
          for k in range(DC // L):
            g[par, i, pl.ds(DC + k * L, L)] = (
                uv * g[par, i, pl.ds(DC + k * L, L)])
        return 0
      lax.fori_loop(0, NUB, row8, 0)

    def pass_body(p, _):
      zero_acc()
      plsc.subcore_barrier()

      def sblk(s, _):
        pltpu.sync_copy(edata_h.at[p, sid, roff + s], ebuf)
        npair = SBB // 2
        dgn, dun = [None] * npair, [None] * npair

        def issue0(l):
          dgn[l] = pltpu.async_copy(table_h.at[ebuf.at[4 * l]], g.at[0],
                                    gsem[0])
          dun[l] = pltpu.async_copy(ubv_h.at[sid, roff + s, 2 * l],
                                    ub.at[0], usem[0])
        issue0(0)
        ds_prev = None
        for l in range(npair):
          dgn[l].wait()
          dun[l].wait()
          if ds_prev is not None:
            ds_prev.wait()
          dg1 = pltpu.async_copy(table_h.at[ebuf.at[4 * l + 2]], g.at[1],
                                 gsem[1])
          scale_hi(0)
          du1 = pltpu.async_copy(ubv_h.at[sid, roff + s, 2 * l + 1],
                                 ub.at[1], usem[1])
          ds0 = pltpu.async_copy(g.at[0], acc.at[ebuf.at[4 * l + 1]],
                                 ssem[0], add=True)
          dg1.wait()
          du1.wait()
          scale_hi(1)
          ds_prev = pltpu.async_copy(g.at[1], acc.at[ebuf.at[4 * l + 3]],
                                     ssem[1], add=True)
          if l + 1 < npair:
            issue0(l + 1)
          ds0.wait()
        ds_prev.wait()
        return 0
      lax.fori_loop(0, nsb, sblk, 0)
      plsc.subcore_barrier()

      pltpu.sync_copy(acc.at[pl.ds(r0, RPT)],
                      sa_out.at[cid, p, pl.ds(r0, RPT)])
      plsc.subcore_barrier()
      return 0
    lax.fori_loop(0, nch, pass_body, 0)

    if with_cnt:
      zero_acc()
      ov = jnp.full((L,), 1.0, jnp.float32)

      def orow(i, _):
        for k in range(W2C // L):
          g[0, i, pl.ds(k * L, L)] = ov
        return 0
      lax.fori_loop(0, B, orow, 0)
      plsc.subcore_barrier()

      def csblk(s, _):
        pltpu.sync_copy(edata_h.at[0, sid, roff + s], ebuf)
        d_prev = None
        for b in range(SBB):
          d = pltpu.async_copy(g.at[0], acc.at[ebuf.at[2 * b + 1]],
                               ssem[b % 2], add=True)
          if d_prev is not None:
            d_prev.wait()
          d_prev = d
        d_prev.wait()
        return 0
      lax.fori_loop(0, nsb, csblk, 0)
      plsc.subcore_barrier()
      pltpu.sync_copy(acc.at[pl.ds(r0, RPT)], cnt_out.at[cid, pl.ds(r0, RPT)])

  return body


def _make_tc_layer(nch, d, chunked_out):
  R = 1264
  grid = (NP // R,)

  def body(sa_ref, c_ref, x_ref, wcat_ref, wrcat_ref, b_ref, o_ref):
    cnt = jnp.sum(c_ref[0] + c_ref[1], axis=1) * (1.0 / W2C)
    inv = 1.0 / jnp.maximum(cnt, 1.0)
    conv = jnp.zeros((R, d), jnp.float32)
    root = jnp.zeros((R, d), jnp.float32)
    for c in range(nch):
      sa = sa_ref[0, c] + sa_ref[1, c]
      conv += jnp.dot(sa, wcat_ref[c], preferred_element_type=jnp.float32,
                      precision="highest")
      root += jnp.dot(x_ref[c], wrcat_ref[c], preferred_element_type=jnp.float32,
                      precision="highest")
    res = conv * inv[:, None] + root + b_ref[...]
    if chunked_out:
      for c in range(nch):
        o_ref[c, :, 0:DC] = res[:, c * DC:(c + 1) * DC]
        o_ref[c, :, DC:W2C] = res[:, c * DC:(c + 1) * DC]
    else:
      o_ref[...] = res

  if chunked_out:
    out_shape = jax.ShapeDtypeStruct((nch, NP, W2C), jnp.float32)
    out_spec = pl.BlockSpec((nch, R, W2C), lambda i: (0, i, 0))
  else:
    out_shape = jax.ShapeDtypeStruct((NP, d), jnp.float32)
    out_spec = pl.BlockSpec((R, d), lambda i: (i, 0))

  return pl.pallas_call(
      body,
      grid=grid,
      in_specs=[
          pl.BlockSpec((NC, nch, R, W2C), lambda i: (0, 0, i, 0)),
          pl.BlockSpec((NC, R, W2C), lambda i: (0, i, 0)),
          pl.BlockSpec((nch, R, W2C), lambda i: (0, i, 0)),
          pl.BlockSpec((nch, W2C, d), lambda i: (0, 0, 0)),
          pl.BlockSpec((nch, W2C, d), lambda i: (0, 0, 0)),
          pl.BlockSpec((1, d), lambda i: (0, 0)),
      ],
      out_specs=out_spec,
      out_shape=out_shape,
  )


def kernel(t, x, edge_index, edge_attr, W1, Wr1, b1, W2, Wr2, b2):
  del t
  n, d = x.shape
  e = edge_index.shape[1]
  nch = d // DC
  epad = NS * SBT * SBB * B

  u = jnp.clip(edge_attr[:, 0], 0.0, 1.0)
  pad = epad - e
  src_p = jnp.concatenate([edge_index[0], jnp.zeros((pad,), jnp.int32)])
  dst_p = jnp.concatenate([edge_index[1], jnp.full((pad,), NP - 1, jnp.int32)])
  u_p = jnp.concatenate([u, jnp.zeros((pad,), jnp.float32)])
  s_r = src_p.reshape(NS, SBT, SBB, B)
  d_r = dst_p.reshape(NS, SBT, SBB, B)
  edata = jnp.stack(
      [jnp.stack([s_r + p * NP, d_r], axis=3) for p in range(nch)],
      axis=0).reshape(nch, NS, SBT, 2 * SBB, B)
  ubv = jnp.broadcast_to(u_p.reshape(NS, SBT, SBB, B, 1),
                         (NS, SBT, SBB, B, L)).reshape(NS, SBT, SBB, NUB, 128)

  xr = x.reshape(n, nch, DC).transpose(1, 0, 2)
  xrp = jnp.pad(xr, ((0, 0), (0, NP - n), (0, 0)))
  tab1 = jnp.concatenate([xrp, xrp], axis=-1)

  def wcat(w0, w1):
    return jnp.concatenate([w0.reshape(nch, DC, d),
                            (w1 - w0).reshape(nch, DC, d)], axis=1)

  def wrcat(wr):
    wrr = wr.reshape(nch, DC, d)
    return jnp.concatenate([wrr, jnp.zeros_like(wrr)], axis=1)

  sc1 = _make_sc_pass(nch, with_cnt=True)
  sc2 = _make_sc_pass(nch, with_cnt=False)
  tc1 = _make_tc_layer(nch, d, chunked_out=True)
  tc2 = _make_tc_layer(nch, d, chunked_out=False)

  zrs = jnp.zeros((RPT, W2C), jnp.float32)
  sa1, cnt = sc1(tab1.reshape(nch * NP, W2C), edata, ubv, zrs)
  hr = tc1(sa1, cnt, tab1, wcat(W1[0], W1[1]), wrcat(Wr1), b1.reshape(1, d))
  (sa2,) = sc2(hr.reshape(nch * NP, W2C), edata, ubv, zrs)
  out = tc2(sa2, cnt, hr, wcat(W2[0], W2[1]), wrcat(Wr2), b2.reshape(1, d))
  return out[:n]


# speedup vs baseline: 1.4129x; 1.0420x over previous
import functools

import jax
import jax.numpy as jnp
from jax import lax
from jax.experimental import pallas as pl
from jax.experimental.pallas import tpu as pltpu
from jax.experimental.pallas import tpu_sc as plsc

NC = 2
NS = 16
L = 16
NW = NC * NS

B = 128
DC = 64
W2C = 2 * DC
NP = 10112
RPT = NP // NS
NUB = B * L // 128

SBB = 8
SB0 = 7
SB1 = 3
SBT = SB0 + SB1


def _make_sc_pass(nch, with_cnt):
  mesh = plsc.VectorSubcoreMesh(core_axis_name="c", subcore_axis_name="s",
                                num_cores=NC, num_subcores=NS)
  out_type = [jax.ShapeDtypeStruct((NC, nch, NP, W2C), jnp.float32)]
  scratch = [
      pltpu.VMEM((2 * SBB, B), jnp.int32),
      pltpu.VMEM((2, B, W2C), jnp.float32),
      pltpu.VMEM((2, NUB, 128), jnp.float32),
      pltpu.VMEM_SHARED((NP, W2C), jnp.float32),
      pltpu.SemaphoreType.DMA,
      pltpu.SemaphoreType.DMA,
      pltpu.SemaphoreType.DMA,
      pltpu.SemaphoreType.DMA,
      pltpu.SemaphoreType.DMA,
      pltpu.SemaphoreType.DMA,
  ]
  if with_cnt:
    out_type.append(jax.ShapeDtypeStruct((NC, NP, W2C), jnp.float32))

  @functools.partial(pl.kernel, out_type=tuple(out_type), mesh=mesh,
                     scratch_types=scratch)
  def body(table_h, edata_h, ubv_h, zrs_h, *rest):
    if with_cnt:
      sa_out, cnt_out, ebuf, g, ub, acc, gs0, gs1, us0, us1, ss0, ss1 = rest
    else:
      sa_out, ebuf, g, ub, acc, gs0, gs1, us0, us1, ss0, ss1 = rest
    gsem, usem, ssem = (gs0, gs1), (us0, us1), (ss0, ss1)
    cid = lax.axis_index("c")
    sid = lax.axis_index("s")
    r0 = sid * RPT
    roff = cid * SB0
    nsb = jnp.where(cid == 0, SB0, SB1)

    def zero_acc():
      pltpu.sync_copy(zrs_h, acc.at[pl.ds(r0, RPT)])

    def scale_hi(par):
      nspl = 128 // L

      def row8(i8, _):
        for sub in range(nspl):
          uv = ub[par, i8, pl.ds(sub * L, L)]
          i = i8 * nspl + sub
          for k in range(DC // L):
            g[par, i, pl.ds(DC + k * L, L)] = (
                uv * g[par, i, pl.ds(DC + k * L, L)])
        return 0
      lax.fori_loop(0, NUB, row8, 0)

    def pass_body(p, _):
      zero_acc()
      plsc.subcore_barrier()

      def sblk(s, _):
        pltpu.sync_copy(edata_h.at[p, sid, roff + s], ebuf)
        npair = SBB // 2
        dgn, dun = [None] * npair, [None] * npair

        def issue0(l):
          dgn[l] = pltpu.async_copy(table_h.at[ebuf.at[4 * l]], g.at[0],
                                    gsem[0])
          dun[l] = pltpu.async_copy(ubv_h.at[sid, roff + s, 2 * l],
                                    ub.at[0], usem[0])
        issue0(0)
        ds_prev = None
        for l in range(npair):
          dgn[l].wait()
          dun[l].wait()
          if ds_prev is not None:
            ds_prev.wait()
          dg1 = pltpu.async_copy(table_h.at[ebuf.at[4 * l + 2]], g.at[1],
                                 gsem[1])
          scale_hi(0)
          du1 = pltpu.async_copy(ubv_h.at[sid, roff + s, 2 * l + 1],
                                 ub.at[1], usem[1])
          ds0 = pltpu.async_copy(g.at[0], acc.at[ebuf.at[4 * l + 1]],
                                 ssem[0], add=True)
          dg1.wait()
          du1.wait()
          scale_hi(1)
          ds_prev = pltpu.async_copy(g.at[1], acc.at[ebuf.at[4 * l + 3]],
                                     ssem[1], add=True)
          if l + 1 < npair:
            issue0(l + 1)
          ds0.wait()
        ds_prev.wait()
        return 0
      lax.fori_loop(0, nsb, sblk, 0)
      plsc.subcore_barrier()

      pltpu.sync_copy(acc.at[pl.ds(r0, RPT)],
                      sa_out.at[cid, p, pl.ds(r0, RPT)])
      plsc.subcore_barrier()
      return 0
    lax.fori_loop(0, nch, pass_body, 0)

    if with_cnt:
      zero_acc()
      ov = jnp.full((L,), 1.0, jnp.float32)

      def orow(i, _):
        for k in range(W2C // L):
          g[0, i, pl.ds(k * L, L)] = ov
        return 0
      lax.fori_loop(0, B, orow, 0)
      plsc.subcore_barrier()

      def csblk(s, _):
        pltpu.sync_copy(edata_h.at[0, sid, roff + s], ebuf)
        d_prev = None
        for b in range(SBB):
          d = pltpu.async_copy(g.at[0], acc.at[ebuf.at[2 * b + 1]],
                               ssem[b % 2], add=True)
          if d_prev is not None:
            d_prev.wait()
          d_prev = d
        d_prev.wait()
        return 0
      lax.fori_loop(0, nsb, csblk, 0)
      plsc.subcore_barrier()
      pltpu.sync_copy(acc.at[pl.ds(r0, RPT)], cnt_out.at[cid, pl.ds(r0, RPT)])

  return body


def _make_tc_layer(nch, d, chunked_out):
  R = 1264
  grid = (NP // R,)

  def body(sa_ref, c_ref, x_ref, wcat_ref, wrcat_ref, b_ref, o_ref):
    cnt = jnp.sum(c_ref[0] + c_ref[1], axis=1) * (1.0 / W2C)
    inv = 1.0 / jnp.maximum(cnt, 1.0)
    conv = jnp.zeros((R, d), jnp.float32)
    root = jnp.zeros((R, d), jnp.float32)
    for c in range(nch):
      sa = sa_ref[0, c] + sa_ref[1, c]
      conv += jnp.dot(sa, wcat_ref[c], preferred_element_type=jnp.float32,
                      precision="highest")
      root += jnp.dot(x_ref[c], wrcat_ref[c], preferred_element_type=jnp.float32,
                      precision="highest")
    res = conv * inv[:, None] + root + b_ref[...]
    if chunked_out:
      for c in range(nch):
        o_ref[c, :, 0:DC] = res[:, c * DC:(c + 1) * DC]
        o_ref[c, :, DC:W2C] = res[:, c * DC:(c + 1) * DC]
    else:
      o_ref[...] = res

  if chunked_out:
    out_shape = jax.ShapeDtypeStruct((nch, NP, W2C), jnp.float32)
    out_spec = pl.BlockSpec((nch, R, W2C), lambda i: (0, i, 0))
  else:
    out_shape = jax.ShapeDtypeStruct((NP, d), jnp.float32)
    out_spec = pl.BlockSpec((R, d), lambda i: (i, 0))

  return pl.pallas_call(
      body,
      grid=grid,
      in_specs=[
          pl.BlockSpec((NC, nch, R, W2C), lambda i: (0, 0, i, 0)),
          pl.BlockSpec((NC, R, W2C), lambda i: (0, i, 0)),
          pl.BlockSpec((nch, R, W2C), lambda i: (0, i, 0)),
          pl.BlockSpec((nch, W2C, d), lambda i: (0, 0, 0)),
          pl.BlockSpec((nch, W2C, d), lambda i: (0, 0, 0)),
          pl.BlockSpec((1, d), lambda i: (0, 0)),
      ],
      out_specs=out_spec,
      out_shape=out_shape,
  )


def kernel(t, x, edge_index, edge_attr, W1, Wr1, b1, W2, Wr2, b2):
  del t
  n, d = x.shape
  e = edge_index.shape[1]
  nch = d // DC
  epad = NS * SBT * SBB * B

  u = jnp.clip(edge_attr[:, 0], 0.0, 1.0)
  pad = epad - e
  src_p = jnp.concatenate([edge_index[0], jnp.zeros((pad,), jnp.int32)])
  dst_p = jnp.concatenate([edge_index[1], jnp.full((pad,), NP - 1, jnp.int32)])
  u_p = jnp.concatenate([u, jnp.zeros((pad,), jnp.float32)])
  s_r = src_p.reshape(NS, SBT, SBB, B)
  d_r = dst_p.reshape(NS, SBT, SBB, B)
  edata = jnp.stack(
      [jnp.stack([s_r + p * NP, d_r], axis=3) for p in range(nch)],
      axis=0).reshape(nch, NS, SBT, 2 * SBB, B)
  ubv = jnp.broadcast_to(u_p.reshape(NS, SBT, SBB, B, 1),
                         (NS, SBT, SBB, B, L)).reshape(NS, SBT, SBB, NUB, 128)

  xr = x.reshape(n, nch, DC).transpose(1, 0, 2)
  xrp = jnp.pad(xr, ((0, 0), (0, NP - n), (0, 0)))
  tab1 = jnp.concatenate([xrp, xrp], axis=-1)

  def wcat(w0, w1):
    return jnp.concatenate([w0.reshape(nch, DC, d),
                            (w1 - w0).reshape(nch, DC, d)], axis=1)

  def wrcat(wr):
    wrr = wr.reshape(nch, DC, d)
    return jnp.concatenate([wrr, jnp.zeros_like(wrr)], axis=1)

  sc1 = _make_sc_pass(nch, with_cnt=True)
  sc2 = _make_sc_pass(nch, with_cnt=False)
  tc1 = _make_tc_layer(nch, d, chunked_out=True)
  tc2 = _make_tc_layer(nch, d, chunked_out=False)

  zrs = jnp.zeros((RPT, W2C), jnp.float32)
  sa1, cnt = sc1(tab1.reshape(nch * NP, W2C), edata, ubv, zrs)
  hr = tc1(sa1, cnt, tab1, wcat(W1[0], W1[1]), wrcat(Wr1), b1.reshape(1, d))
  (sa2,) = sc2(hr.reshape(nch * NP, W2C), edata, ubv, zrs)
  out = tc2(sa2, cnt, hr, wcat(W2[0], W2[1]), wrcat(Wr2), b2.reshape(1, d))
  return out[:n]


# speedup vs baseline: 1.4859x; 1.0517x over previous
import functools

import jax
import jax.numpy as jnp
from jax import lax
from jax.experimental import pallas as pl
from jax.experimental.pallas import tpu as pltpu
from jax.experimental.pallas import tpu_sc as plsc

NC = 2
NS = 16
L = 16
NW = NC * NS

B = 128
DC = 64
W2C = 2 * DC
NP = 10112
RPT = NP // NS
NUB = B * L // 128

SBB = 8
SB0 = 7
SB1 = 3
SBT = SB0 + SB1


def _make_sc_pass(nch, with_cnt):
  mesh = plsc.VectorSubcoreMesh(core_axis_name="c", subcore_axis_name="s",
                                num_cores=NC, num_subcores=NS)
  out_type = [jax.ShapeDtypeStruct((NC, nch, NP, W2C), jnp.float32)]
  scratch = [
      pltpu.VMEM((2 * SBB, B), jnp.int32),
      pltpu.VMEM((2, B, W2C), jnp.float32),
      pltpu.VMEM((2, NUB, 128), jnp.float32),
      pltpu.VMEM_SHARED((NP, W2C), jnp.float32),
      pltpu.SemaphoreType.DMA,
      pltpu.SemaphoreType.DMA,
      pltpu.SemaphoreType.DMA,
      pltpu.SemaphoreType.DMA,
      pltpu.SemaphoreType.DMA,
      pltpu.SemaphoreType.DMA,
  ]
  if with_cnt:
    out_type.append(jax.ShapeDtypeStruct((NC, NP, W2C), jnp.float32))

  @functools.partial(pl.kernel, out_type=tuple(out_type), mesh=mesh,
                     scratch_types=scratch)
  def body(table_h, edata_h, ubv_h, zrs_h, *rest):
    if with_cnt:
      sa_out, cnt_out, ebuf, g, ub, acc, gs0, gs1, us0, us1, ss0, ss1 = rest
    else:
      sa_out, ebuf, g, ub, acc, gs0, gs1, us0, us1, ss0, ss1 = rest
    gsem, usem, ssem = (gs0, gs1), (us0, us1), (ss0, ss1)
    cid = lax.axis_index("c")
    sid = lax.axis_index("s")
    r0 = sid * RPT
    roff = cid * SB0
    nsb = jnp.where(cid == 0, SB0, SB1)

    def zero_acc():
      pltpu.sync_copy(zrs_h, acc.at[pl.ds(r0, RPT)])

    def scale_hi(par):
      nspl = 128 // L

      def row8(i8, _):
        for sub in range(nspl):
          uv = ub[par, i8, pl.ds(sub * L, L)]
          i = i8 * nspl + sub
          for k in range(DC // L):
            g[par, i, pl.ds(DC + k * L, L)] = (
                uv * g[par, i, pl.ds(DC + k * L, L)])
        return 0
      lax.fori_loop(0, NUB, row8, 0)

    def pass_body(p, _):
      zero_acc()
      plsc.subcore_barrier()

      def sblk(s, _):
        pltpu.sync_copy(edata_h.at[p, sid, roff + s], ebuf)
        npair = SBB // 2
        dgn, dun = [None] * npair, [None] * npair

        def issue0(l):
          dgn[l] = pltpu.async_copy(table_h.at[ebuf.at[4 * l]], g.at[0],
                                    gsem[0])
          dun[l] = pltpu.async_copy(ubv_h.at[sid, roff + s, 2 * l],
                                    ub.at[0], usem[0])
        issue0(0)
        ds_prev = None
        for l in range(npair):
          dgn[l].wait()
          dun[l].wait()
          if ds_prev is not None:
            ds_prev.wait()
          dg1 = pltpu.async_copy(table_h.at[ebuf.at[4 * l + 2]], g.at[1],
                                 gsem[1])
          scale_hi(0)
          du1 = pltpu.async_copy(ubv_h.at[sid, roff + s, 2 * l + 1],
                                 ub.at[1], usem[1])
          ds0 = pltpu.async_copy(g.at[0], acc.at[ebuf.at[4 * l + 1]],
                                 ssem[0], add=True)
          dg1.wait()
          du1.wait()
          scale_hi(1)
          ds_prev = pltpu.async_copy(g.at[1], acc.at[ebuf.at[4 * l + 3]],
                                     ssem[1], add=True)
          if l + 1 < npair:
            issue0(l + 1)
          ds0.wait()
        ds_prev.wait()
        return 0
      lax.fori_loop(0, nsb, sblk, 0)
      plsc.subcore_barrier()

      pltpu.sync_copy(acc.at[pl.ds(r0, RPT)],
                      sa_out.at[cid, p, pl.ds(r0, RPT)])
      plsc.subcore_barrier()
      return 0
    lax.fori_loop(0, nch, pass_body, 0)

    if with_cnt:
      zero_acc()
      ov = jnp.full((L,), 1.0, jnp.float32)

      def orow(i, _):
        for k in range(W2C // L):
          g[0, i, pl.ds(k * L, L)] = ov
        return 0
      lax.fori_loop(0, B, orow, 0)
      plsc.subcore_barrier()

      def csblk(s, _):
        pltpu.sync_copy(edata_h.at[0, sid, s], ebuf)
        d_prev = None
        for b in range(SBB):
          d = pltpu.async_copy(g.at[0], acc.at[ebuf.at[2 * b + 1]],
                               ssem[b % 2], add=True)
          if d_prev is not None:
            d_prev.wait()
          d_prev = d
        d_prev.wait()
        return 0
      lax.fori_loop(0, jnp.where(cid == 0, SBT, 0), csblk, 0)
      plsc.subcore_barrier()
      pltpu.sync_copy(acc.at[pl.ds(r0, RPT)], cnt_out.at[cid, pl.ds(r0, RPT)])

  return body


def _make_tc_layer(nch, d, chunked_out):
  R = 1264
  grid = (NP // R,)

  def body(sa_ref, c_ref, x_ref, wcat_ref, wrcat_ref, b_ref, o_ref):
    cnt = jnp.sum(c_ref[0] + c_ref[1], axis=1) * (1.0 / W2C)
    inv = 1.0 / jnp.maximum(cnt, 1.0)
    conv = jnp.zeros((R, d), jnp.float32)
    root = jnp.zeros((R, d), jnp.float32)
    for c in range(nch):
      sa = sa_ref[0, c] + sa_ref[1, c]
      conv += jnp.dot(sa, wcat_ref[c], preferred_element_type=jnp.float32)
      root += jnp.dot(x_ref[c], wrcat_ref[c], preferred_element_type=jnp.float32)
    res = conv * inv[:, None] + root + b_ref[...]
    if chunked_out:
      for c in range(nch):
        o_ref[c, :, 0:DC] = res[:, c * DC:(c + 1) * DC]
        o_ref[c, :, DC:W2C] = res[:, c * DC:(c + 1) * DC]
    else:
      o_ref[...] = res

  if chunked_out:
    out_shape = jax.ShapeDtypeStruct((nch, NP, W2C), jnp.float32)
    out_spec = pl.BlockSpec((nch, R, W2C), lambda i: (0, i, 0))
  else:
    out_shape = jax.ShapeDtypeStruct((NP, d), jnp.float32)
    out_spec = pl.BlockSpec((R, d), lambda i: (i, 0))

  return pl.pallas_call(
      body,
      grid=grid,
      in_specs=[
          pl.BlockSpec((NC, nch, R, W2C), lambda i: (0, 0, i, 0)),
          pl.BlockSpec((NC, R, W2C), lambda i: (0, i, 0)),
          pl.BlockSpec((nch, R, W2C), lambda i: (0, i, 0)),
          pl.BlockSpec((nch, W2C, d), lambda i: (0, 0, 0)),
          pl.BlockSpec((nch, W2C, d), lambda i: (0, 0, 0)),
          pl.BlockSpec((1, d), lambda i: (0, 0)),
      ],
      out_specs=out_spec,
      out_shape=out_shape,
  )


def kernel(t, x, edge_index, edge_attr, W1, Wr1, b1, W2, Wr2, b2):
  del t
  n, d = x.shape
  e = edge_index.shape[1]
  nch = d // DC
  epad = NS * SBT * SBB * B

  u = jnp.clip(edge_attr[:, 0], 0.0, 1.0)
  pad = epad - e
  src_p = jnp.concatenate([edge_index[0], jnp.zeros((pad,), jnp.int32)])
  dst_p = jnp.concatenate([edge_index[1], jnp.full((pad,), NP - 1, jnp.int32)])
  u_p = jnp.concatenate([u, jnp.zeros((pad,), jnp.float32)])
  s_r = src_p.reshape(NS, SBT, SBB, B)
  d_r = dst_p.reshape(NS, SBT, SBB, B)
  edata = jnp.stack(
      [jnp.stack([s_r + p * NP, d_r], axis=3) for p in range(nch)],
      axis=0).reshape(nch, NS, SBT, 2 * SBB, B)
  ubv = jnp.broadcast_to(u_p.reshape(NS, SBT, SBB, B, 1),
                         (NS, SBT, SBB, B, L)).reshape(NS, SBT, SBB, NUB, 128)

  xr = x.reshape(n, nch, DC).transpose(1, 0, 2)
  xrp = jnp.pad(xr, ((0, 0), (0, NP - n), (0, 0)))
  tab1 = jnp.concatenate([xrp, xrp], axis=-1)

  def wcat(w0, w1):
    return jnp.concatenate([w0.reshape(nch, DC, d),
                            (w1 - w0).reshape(nch, DC, d)], axis=1)

  def wrcat(wr):
    wrr = wr.reshape(nch, DC, d)
    return jnp.concatenate([wrr, jnp.zeros_like(wrr)], axis=1)

  sc1 = _make_sc_pass(nch, with_cnt=True)
  sc2 = _make_sc_pass(nch, with_cnt=False)
  tc1 = _make_tc_layer(nch, d, chunked_out=True)
  tc2 = _make_tc_layer(nch, d, chunked_out=False)

  zrs = jnp.zeros((RPT, W2C), jnp.float32)
  sa1, cnt = sc1(tab1.reshape(nch * NP, W2C), edata, ubv, zrs)
  hr = tc1(sa1, cnt, tab1, wcat(W1[0], W1[1]), wrcat(Wr1), b1.reshape(1, d))
  (sa2,) = sc2(hr.reshape(nch * NP, W2C), edata, ubv, zrs)
  out = tc2(sa2, cnt, hr, wcat(W2[0], W2[1]), wrcat(Wr2), b2.reshape(1, d))
  return out[:n]
